# Initial kernel scaffold; baseline (speedup 1.0000x reference)
import jax
import jax.numpy as jnp
from jax.experimental import pallas as pl


def kernel(x):
    raise NotImplementedError("write your pallas kernel here")


# speedup vs baseline: 3.3751x; 3.3751x over previous
import jax
import jax.numpy as jnp
import numpy as np
from jax import lax
from jax.experimental import pallas as pl
from jax.experimental.pallas import tpu as pltpu
from jax.experimental.pallas import tpu_sc as plsc

B = 128
N = 8192
K = 32
L = 16
NC = 2
NS = 16
NW = NC * NS
ROWS_PER_W = B // NW
NV = N // L

INT_MIN = np.int32(-2147483648)
MASK31 = np.int32(0x7FFFFFFF)


def _digit(ub, shift):
    return lax.shift_right_logical(ub, np.int32(shift)) & np.int32(0xF)


def _sc_body(x_hbm, out_hbm, xv, kv, c0, c1, hist):
    wid = lax.axis_index("s") * NC + lax.axis_index("c")
    lane = lax.iota(jnp.int32, L)
    lane16 = lane * L
    ones = jnp.ones((L,), jnp.int32)
    zeros16 = jnp.zeros((L,), jnp.int32)

    for j in range(ROWS_PER_W):
        row = wid * ROWS_PER_W + j
        pltpu.sync_copy(x_hbm.at[row], xv)

        for l in range(L):
            hist[pl.ds(l * L, L)] = zeros16

        def l0_body(i, carry):
            v = xv[pl.ds(i * L, L)]
            bb = lax.bitcast_convert_type(v, jnp.int32)
            key = bb ^ (lax.shift_right_arithmetic(bb, 31) & MASK31)
            ub = key ^ INT_MIN
            kv[pl.ds(i * L, L)] = ub
            d = _digit(ub, 28)
            plsc.addupdate_scatter(hist, [lane16 + d], ones)
            return carry

        lax.fori_loop(0, NV, l0_body, 0)

        def pick_bucket(kr):
            comb = hist[pl.ds(0, L)]
            for l in range(1, L):
                comb = comb + hist[pl.ds(l * L, L)]
            suf = jnp.flip(plsc.cumsum(jnp.flip(comb)))
            ge = suf >= kr
            bsel = jnp.sum(ge.astype(jnp.int32)) - 1
            c_above = jnp.sum(jnp.where(ge, 0, comb))
            mnew = jnp.sum(jnp.where(lane == bsel, comb, 0))
            return bsel, kr - c_above, mnew

        kr = np.int32(K)
        bsel, kr, m = pick_bucket(kr)
        tkey = lax.shift_left(bsel, 28)

        def compact0_body(i, off):
            ub = kv[pl.ds(i * L, L)]
            msk = _digit(ub, 28) == bsel
            plsc.store_compressed(c0.at[pl.ds(off, L)], ub, mask=msk)
            return off + jnp.sum(msk.astype(jnp.int32))

        lax.fori_loop(0, NV, compact0_body, np.int32(0))

        src, dst = c0, c1
        for shift in range(24, -1, -4):
            for l in range(L):
                hist[pl.ds(l * L, L)] = zeros16
            nv = lax.shift_right_logical(m + np.int32(L - 1), np.int32(4))

            def hist_body(i, carry, src=src, m=m, shift=shift):
                ub = src[pl.ds(i * L, L)]
                valid = lane < (m - i * L)
                d = _digit(ub, shift)
                plsc.addupdate_scatter(hist, [lane16 + d], ones, mask=valid)
                return carry

            lax.fori_loop(0, nv, hist_body, 0)
            bsel, kr, m2 = pick_bucket(kr)
            tkey = tkey | lax.shift_left(bsel, shift)

            if shift > 0:
                def compact_body(i, off, src=src, dst=dst, m=m, shift=shift,
                                 bsel=bsel):
                    ub = src[pl.ds(i * L, L)]
                    valid = lane < (m - i * L)
                    msk = valid & (_digit(ub, shift) == bsel)
                    plsc.store_compressed(dst.at[pl.ds(off, L)], ub, mask=msk)
                    return off + jnp.sum(msk.astype(jnp.int32))

                lax.fori_loop(0, nv, compact_body, np.int32(0))
                src, dst = dst, src
            m = m2

        key_t = tkey ^ INT_MIN
        fb = key_t ^ (lax.shift_right_arithmetic(key_t, 31) & MASK31)
        tvec = lax.bitcast_convert_type(jnp.full((L,), fb, jnp.int32), jnp.float32)

        def mask_body(i, carry):
            v = xv[pl.ds(i * L, L)]
            xv[pl.ds(i * L, L)] = jnp.where(v >= tvec, v, np.float32(0.0))
            return carry

        lax.fori_loop(0, NV, mask_body, 0)
        pltpu.sync_copy(xv, out_hbm.at[row])


@jax.jit
def kernel(x):
    mesh = plsc.VectorSubcoreMesh(
        core_axis_name="c", subcore_axis_name="s", num_cores=NC,
        num_subcores=NS)
    return pl.kernel(
        _sc_body,
        out_type=jax.ShapeDtypeStruct((B, N), jnp.float32),
        mesh=mesh,
        compiler_params=pltpu.CompilerParams(needs_layout_passes=False),
        scratch_types=[
            pltpu.VMEM((N,), jnp.float32),
            pltpu.VMEM((N,), jnp.int32),
            pltpu.VMEM((N + L,), jnp.int32),
            pltpu.VMEM((N + L,), jnp.int32),
            pltpu.VMEM((L * L,), jnp.int32),
        ],
    )(x)
